# Initial kernel scaffold; baseline (speedup 1.0000x reference)
import jax
import jax.numpy as jnp
from jax.experimental import pallas as pl


def kernel(x, edge_index, W, b, gamma, beta):
    raise NotImplementedError("write your pallas kernel here")


# speedup vs baseline: 11.1575x; 11.1575x over previous
import functools

import jax
import jax.numpy as jnp
from jax import lax
from jax.experimental import pallas as pl
from jax.experimental.pallas import tpu as pltpu
from jax.experimental.pallas import tpu_sc as plsc

_N = 10000
_E = 320000
_D = 128
_NC = 2
_NS = 16
_NW = _NC * _NS
_B = 128
_NB = 80
_EP = _NW * _NB * _B
_NP = 10240
_RPW = _NP // _NS

_mesh = plsc.VectorSubcoreMesh(core_axis_name="c", subcore_axis_name="s")


@functools.partial(
    pl.kernel,
    out_type=jax.ShapeDtypeStruct((_NC, _NP, _D), jnp.float32),
    mesh=_mesh,
    scratch_types=[
        pltpu.VMEM((_NB, _B), jnp.int32),
        pltpu.VMEM((_B, _D), jnp.float32),
        pltpu.VMEM((_B, _D), jnp.float32),
        pltpu.VMEM_SHARED((_NP, _D), jnp.float32),
    ],
)
def _sc_deg(dst_hbm, zrows_hbm, ones_hbm, deg_hbm, idx_v, stg_v, ones_v, deg_s):
    cid = lax.axis_index("c")
    sid = lax.axis_index("s")
    wid = cid * _NS + sid
    pltpu.sync_copy(dst_hbm.at[wid], idx_v)
    pltpu.sync_copy(ones_hbm, ones_v)
    pltpu.sync_copy(zrows_hbm, stg_v)
    for k in range(_RPW // _B):
        pltpu.sync_copy(stg_v, deg_s.at[pl.ds(sid * _RPW + k * _B, _B)])
    plsc.subcore_barrier()

    def step(j, carry):
        pltpu.sync_copy(ones_v, deg_s.at[idx_v.at[j]], add=True)
        return carry

    lax.fori_loop(0, _NB, step, 0)
    plsc.subcore_barrier()
    for k in range(_RPW // _B):
        pltpu.sync_copy(deg_s.at[pl.ds(sid * _RPW + k * _B, _B)], stg_v)
        pltpu.sync_copy(stg_v, deg_hbm.at[cid, pl.ds(sid * _RPW + k * _B, _B)])


@functools.partial(
    pl.kernel,
    out_type=jax.ShapeDtypeStruct((_NC, _NP, _D), jnp.float32),
    mesh=_mesh,
    scratch_types=[
        pltpu.VMEM((_NB, _B), jnp.int32),
        pltpu.VMEM((_NB, _B), jnp.int32),
        pltpu.VMEM((_B, _D), jnp.float32),
        pltpu.VMEM_SHARED((_NP, _D), jnp.float32),
        pltpu.SemaphoreType.DMA,
    ],
)
def _sc_agg(g_hbm, src_hbm, dst_hbm, zrows_hbm, agg_hbm,
            sidx_v, didx_v, rows_v, agg_s, sem):
    cid = lax.axis_index("c")
    sid = lax.axis_index("s")
    wid = cid * _NS + sid
    pltpu.sync_copy(src_hbm.at[wid], sidx_v)
    pltpu.sync_copy(dst_hbm.at[wid], didx_v)
    pltpu.sync_copy(zrows_hbm, rows_v)
    for k in range(_RPW // _B):
        pltpu.sync_copy(rows_v, agg_s.at[pl.ds(sid * _RPW + k * _B, _B)])
    plsc.subcore_barrier()

    def step(j, carry):
        pltpu.async_copy(g_hbm.at[sidx_v.at[j]], rows_v, sem).wait()
        pltpu.sync_copy(rows_v, agg_s.at[didx_v.at[j]], add=True)
        return carry

    lax.fori_loop(0, _NB, step, 0)
    plsc.subcore_barrier()
    for k in range(_RPW // _B):
        pltpu.sync_copy(agg_s.at[pl.ds(sid * _RPW + k * _B, _B)], rows_v)
        pltpu.sync_copy(rows_v, agg_hbm.at[cid, pl.ds(sid * _RPW + k * _B, _B)])


def _prep_body(x_ref, w_ref, deg_ref, g_ref, dis_ref):
    deg = deg_ref[0, 0:_N, 0:1] + deg_ref[1, 0:_N, 0:1] + 1.0
    dis = lax.rsqrt(deg)
    h = jnp.dot(x_ref[...], w_ref[...], preferred_element_type=jnp.float32)
    g_ref[...] = h * dis
    dis_ref[...] = dis


def _tc_prep(x, W, deg128):
    return pl.pallas_call(
        _prep_body,
        out_shape=[
            jax.ShapeDtypeStruct((_N, _D), jnp.float32),
            jax.ShapeDtypeStruct((_N, 1), jnp.float32),
        ],
    )(x, W, deg128)


def _final_body(agg_ref, g_ref, dis_ref, b_ref, gam_ref, bet_ref, o_ref):
    pre = (agg_ref[0, 0:_N, :] + agg_ref[1, 0:_N, :] + g_ref[...]) * dis_ref[...] + b_ref[...]
    mean = jnp.mean(pre, axis=0, keepdims=True)
    cen = pre - mean
    var = jnp.mean(cen * cen, axis=0, keepdims=True)
    o_ref[...] = jnp.maximum(
        cen * lax.rsqrt(var + 1e-5) * gam_ref[...] + bet_ref[...], 0.0)


def _tc_final(agg, g, dis, b, gamma, beta):
    return pl.pallas_call(
        _final_body,
        out_shape=jax.ShapeDtypeStruct((_N, _D), jnp.float32),
    )(agg, g, dis, b, gamma, beta)


def kernel(x, edge_index, W, b, gamma, beta):
    pad = _EP - _E
    src = jnp.concatenate(
        [edge_index[0], jnp.zeros((pad,), jnp.int32)]).reshape(_NW, _NB, _B)
    dst = jnp.concatenate(
        [edge_index[1], jnp.full((pad,), _NP - 1, jnp.int32)]).reshape(_NW, _NB, _B)
    zrows = jnp.zeros((_B, _D), jnp.float32)
    ones = jnp.ones((_B, _D), jnp.float32)
    deg128 = _sc_deg(dst, zrows, ones)
    g, dis = _tc_prep(x, W, deg128)
    agg = _sc_agg(g, src, dst, zrows)
    out = _tc_final(agg, g, dis, b.reshape(1, _D), gamma.reshape(1, _D),
                    beta.reshape(1, _D))
    return out


# speedup vs baseline: 28.2016x; 2.5276x over previous
import functools

import jax
import jax.numpy as jnp
from jax import lax
from jax.experimental import pallas as pl
from jax.experimental.pallas import tpu as pltpu
from jax.experimental.pallas import tpu_sc as plsc

_N = 10000
_E = 320000
_D = 128
_NC = 2
_NS = 16
_NW = _NC * _NS
_B = 128
_NB = 80
_EP = _NW * _NB * _B
_NP = 10240
_RPW = _NP // _NS
_L = 16

_mesh = plsc.VectorSubcoreMesh(core_axis_name="c", subcore_axis_name="s")


@functools.partial(
    pl.kernel,
    out_type=jax.ShapeDtypeStruct((_NC, _NP, _D), jnp.float32),
    mesh=_mesh,
    scratch_types=[
        pltpu.VMEM((_NB, _B), jnp.int32),
        pltpu.VMEM((_B, _D), jnp.float32),
        pltpu.VMEM((_B, _D), jnp.float32),
        pltpu.VMEM_SHARED((_NP, _D), jnp.float32),
        pltpu.SemaphoreType.DMA,
    ],
)
def _sc_deg(dst_hbm, zrows_hbm, ones_hbm, deg_hbm, idx_v, stg_v, ones_v,
            deg_s, sem):
    cid = lax.axis_index("c")
    sid = lax.axis_index("s")
    wid = cid * _NS + sid
    pltpu.sync_copy(dst_hbm.at[wid], idx_v)
    pltpu.sync_copy(ones_hbm, ones_v)
    pltpu.sync_copy(zrows_hbm, stg_v)
    for k in range(_RPW // _B):
        pltpu.sync_copy(stg_v, deg_s.at[pl.ds(sid * _RPW + k * _B, _B)])
    plsc.subcore_barrier()

    def fire(j, carry):
        pltpu.async_copy(ones_v, deg_s.at[idx_v.at[j]], sem, add=True)
        return carry

    lax.fori_loop(0, _NB, fire, 0)

    def drain(j, carry):
        pltpu.make_async_copy(ones_v, deg_s.at[idx_v.at[j]], sem).wait()
        return carry

    lax.fori_loop(0, _NB, drain, 0)
    plsc.subcore_barrier()
    for k in range(_RPW // _B):
        pltpu.sync_copy(deg_s.at[pl.ds(sid * _RPW + k * _B, _B)], stg_v)
        pltpu.sync_copy(stg_v, deg_hbm.at[cid, pl.ds(sid * _RPW + k * _B, _B)])


@functools.partial(
    pl.kernel,
    out_type=jax.ShapeDtypeStruct((_NC, _NP, _D), jnp.float32),
    mesh=_mesh,
    scratch_types=[
        pltpu.VMEM((_NB, _B), jnp.int32),
        pltpu.VMEM((_B,), jnp.int32),
        pltpu.VMEM((_B,), jnp.int32),
        pltpu.VMEM((_B,), jnp.int32),
        pltpu.VMEM((_B,), jnp.int32),
        pltpu.VMEM((_B, _D), jnp.float32),
        pltpu.VMEM((_B, _D), jnp.float32),
        pltpu.VMEM_SHARED((_NP, _D), jnp.float32),
        pltpu.SemaphoreType.DMA,
        pltpu.SemaphoreType.DMA,
        pltpu.SemaphoreType.DMA,
        pltpu.SemaphoreType.DMA,
    ],
)
def _sc_agg(g_hbm, pidx_hbm, zrows_hbm, agg_hbm,
            pidx_v, si0, si1, di0, di1, rows0, rows1, agg_s,
            gs0, gs1, ss0, ss1):
    cid = lax.axis_index("c")
    sid = lax.axis_index("s")
    wid = cid * _NS + sid
    pltpu.sync_copy(pidx_hbm.at[wid], pidx_v)
    pltpu.sync_copy(zrows_hbm, rows0)
    for k in range(_RPW // _B):
        pltpu.sync_copy(rows0, agg_s.at[pl.ds(sid * _RPW + k * _B, _B)])
    plsc.subcore_barrier()

    def unpack(j, si, di):
        for k in range(_B // _L):
            p = pidx_v[j, pl.ds(k * _L, _L)]
            si[pl.ds(k * _L, _L)] = lax.bitwise_and(p, 0xFFFF)
            di[pl.ds(k * _L, _L)] = lax.shift_right_logical(p, 16)

    unpack(0, si0, di0)
    pltpu.async_copy(g_hbm.at[si0], rows0, gs0)
    unpack(1, si1, di1)
    pltpu.async_copy(g_hbm.at[si1], rows1, gs1)

    def step(i, carry):
        j = 2 * i
        pltpu.make_async_copy(g_hbm.at[si0], rows0, gs0).wait()
        sc0 = pltpu.async_copy(rows0, agg_s.at[di0], ss0, add=True)
        pltpu.make_async_copy(g_hbm.at[si1], rows1, gs1).wait()
        sc1 = pltpu.async_copy(rows1, agg_s.at[di1], ss1, add=True)
        sc0.wait()

        @pl.when(j + 2 < _NB)
        def _():
            unpack(j + 2, si0, di0)
            pltpu.async_copy(g_hbm.at[si0], rows0, gs0)

        sc1.wait()

        @pl.when(j + 3 < _NB)
        def _():
            unpack(j + 3, si1, di1)
            pltpu.async_copy(g_hbm.at[si1], rows1, gs1)

        return carry

    lax.fori_loop(0, _NB // 2, step, 0)
    plsc.subcore_barrier()
    for k in range(_RPW // _B):
        pltpu.sync_copy(agg_s.at[pl.ds(sid * _RPW + k * _B, _B)], rows0)
        pltpu.sync_copy(rows0, agg_hbm.at[cid, pl.ds(sid * _RPW + k * _B, _B)])


def _prep_body(x_ref, w_ref, deg_ref, g_ref, dis_ref):
    deg = deg_ref[0, 0:_N, 0:1] + deg_ref[1, 0:_N, 0:1] + 1.0
    dis = lax.rsqrt(deg)
    h = jnp.dot(x_ref[...], w_ref[...], preferred_element_type=jnp.float32)
    g_ref[...] = h * dis
    dis_ref[...] = dis


def _tc_prep(x, W, deg128):
    return pl.pallas_call(
        _prep_body,
        out_shape=[
            jax.ShapeDtypeStruct((_N, _D), jnp.float32),
            jax.ShapeDtypeStruct((_N, 1), jnp.float32),
        ],
    )(x, W, deg128)


def _final_body(agg_ref, g_ref, dis_ref, b_ref, gam_ref, bet_ref, o_ref):
    pre = (agg_ref[0, 0:_N, :] + agg_ref[1, 0:_N, :] + g_ref[...]) * dis_ref[...] + b_ref[...]
    mean = jnp.mean(pre, axis=0, keepdims=True)
    cen = pre - mean
    var = jnp.mean(cen * cen, axis=0, keepdims=True)
    o_ref[...] = jnp.maximum(
        cen * lax.rsqrt(var + 1e-5) * gam_ref[...] + bet_ref[...], 0.0)


def _tc_final(agg, g, dis, b, gamma, beta):
    return pl.pallas_call(
        _final_body,
        out_shape=jax.ShapeDtypeStruct((_N, _D), jnp.float32),
    )(agg, g, dis, b, gamma, beta)


def kernel(x, edge_index, W, b, gamma, beta):
    pad = _EP - _E
    spread = jnp.arange(pad, dtype=jnp.int32)
    src_pad = spread % _N
    dst_pad = _N + (spread % (_NP - _N))
    src = jnp.concatenate([edge_index[0], src_pad])
    dst = jnp.concatenate([edge_index[1], dst_pad])
    dst_r = dst.reshape(_NW, _NB, _B)
    pidx = (src | (dst << 16)).reshape(_NW, _NB, _B)
    zrows = jnp.zeros((_B, _D), jnp.float32)
    ones = jnp.ones((_B, _D), jnp.float32)
    deg128 = _sc_deg(dst_r, zrows, ones)
    g, dis = _tc_prep(x, W, deg128)
    agg = _sc_agg(g, pidx, zrows)
    out = _tc_final(agg, g, dis, b.reshape(1, _D), gamma.reshape(1, _D),
                    beta.reshape(1, _D))
    return out


# speedup vs baseline: 28.2128x; 1.0004x over previous
import functools

import jax
import jax.numpy as jnp
from jax import lax
from jax.experimental import pallas as pl
from jax.experimental.pallas import tpu as pltpu
from jax.experimental.pallas import tpu_sc as plsc

_N = 10000
_E = 320000
_D = 128
_NC = 2
_NS = 16
_NW = _NC * _NS
_B = 128
_NB = 80
_EP = _NW * _NB * _B
_NP = 10240
_RPW = _NP // _NS
_L = 16

_mesh = plsc.VectorSubcoreMesh(core_axis_name="c", subcore_axis_name="s")


@functools.partial(
    pl.kernel,
    out_type=jax.ShapeDtypeStruct((_NC, _NP, _D), jnp.float32),
    mesh=_mesh,
    scratch_types=[
        pltpu.VMEM((_NB, _B), jnp.int32),
        pltpu.VMEM((_B, _D), jnp.float32),
        pltpu.VMEM((_B, _D), jnp.float32),
        pltpu.VMEM_SHARED((_NP, _D), jnp.float32),
        pltpu.SemaphoreType.DMA,
    ],
)
def _sc_deg(dst_hbm, zrows_hbm, ones_hbm, deg_hbm, idx_v, stg_v, ones_v,
            deg_s, sem):
    cid = lax.axis_index("c")
    sid = lax.axis_index("s")
    wid = cid * _NS + sid
    pltpu.sync_copy(dst_hbm.at[wid], idx_v)
    pltpu.sync_copy(ones_hbm, ones_v)
    pltpu.sync_copy(zrows_hbm, stg_v)
    for k in range(_RPW // _B):
        pltpu.sync_copy(stg_v, deg_s.at[pl.ds(sid * _RPW + k * _B, _B)])
    plsc.subcore_barrier()

    def fire(j, carry):
        pltpu.async_copy(ones_v, deg_s.at[idx_v.at[j]], sem, add=True)
        return carry

    lax.fori_loop(0, _NB, fire, 0)

    def drain(j, carry):
        pltpu.make_async_copy(ones_v, deg_s.at[idx_v.at[j]], sem).wait()
        return carry

    lax.fori_loop(0, _NB, drain, 0)
    plsc.subcore_barrier()
    for k in range(_RPW // _B):
        pltpu.sync_copy(deg_s.at[pl.ds(sid * _RPW + k * _B, _B)], stg_v)
        pltpu.sync_copy(stg_v, deg_hbm.at[cid, pl.ds(sid * _RPW + k * _B, _B)])


@functools.partial(
    pl.kernel,
    out_type=jax.ShapeDtypeStruct((_NC, _NP, _D), jnp.float32),
    mesh=_mesh,
    scratch_types=[
        pltpu.VMEM((_NB, _B), jnp.int32),
        pltpu.VMEM((_B,), jnp.int32),
        pltpu.VMEM((_B,), jnp.int32),
        pltpu.VMEM((_B,), jnp.int32),
        pltpu.VMEM((_B,), jnp.int32),
        pltpu.VMEM((_B, _D), jnp.float32),
        pltpu.VMEM((_B, _D), jnp.float32),
        pltpu.VMEM_SHARED((_NP, _D), jnp.float32),
        pltpu.SemaphoreType.DMA,
        pltpu.SemaphoreType.DMA,
        pltpu.SemaphoreType.DMA,
        pltpu.SemaphoreType.DMA,
    ],
)
def _sc_agg(g_hbm, pidx_hbm, zrows_hbm, agg_hbm,
            pidx_v, si0, si1, di0, di1, rows0, rows1, agg_s,
            gs0, gs1, ss0, ss1):
    cid = lax.axis_index("c")
    sid = lax.axis_index("s")
    wid = cid * _NS + sid
    pltpu.sync_copy(pidx_hbm.at[wid], pidx_v)
    pltpu.sync_copy(zrows_hbm, rows0)
    for k in range(_RPW // _B):
        pltpu.sync_copy(rows0, agg_s.at[pl.ds(sid * _RPW + k * _B, _B)])
    plsc.subcore_barrier()

    def unpack(j, si, di):
        for k in range(_B // _L):
            p = pidx_v[j, pl.ds(k * _L, _L)]
            si[pl.ds(k * _L, _L)] = lax.bitwise_and(p, 0xFFFF)
            di[pl.ds(k * _L, _L)] = lax.shift_right_logical(p, 16)

    unpack(0, si0, di0)
    pltpu.async_copy(g_hbm.at[si0], rows0, gs0)
    unpack(1, si1, di1)
    pltpu.async_copy(g_hbm.at[si1], rows1, gs1)

    def step(i, carry):
        j = 2 * i
        pltpu.make_async_copy(g_hbm.at[si0], rows0, gs0).wait()
        sc0 = pltpu.async_copy(rows0, agg_s.at[di0], ss0, add=True)
        pltpu.make_async_copy(g_hbm.at[si1], rows1, gs1).wait()
        sc1 = pltpu.async_copy(rows1, agg_s.at[di1], ss1, add=True)
        sc0.wait()

        @pl.when(j + 2 < _NB)
        def _():
            unpack(j + 2, si0, di0)
            pltpu.async_copy(g_hbm.at[si0], rows0, gs0)

        sc1.wait()

        @pl.when(j + 3 < _NB)
        def _():
            unpack(j + 3, si1, di1)
            pltpu.async_copy(g_hbm.at[si1], rows1, gs1)

        return carry

    lax.fori_loop(0, _NB // 2, step, 0)
    plsc.subcore_barrier()
    for k in range(_RPW // _B):
        pltpu.sync_copy(agg_s.at[pl.ds(sid * _RPW + k * _B, _B)], rows0)
        pltpu.sync_copy(rows0, agg_hbm.at[cid, pl.ds(sid * _RPW + k * _B, _B)])


def _prep_body(x_ref, w_ref, deg_ref, g_ref, dis_ref):
    deg = deg_ref[0, 0:_N, 0:1] + deg_ref[1, 0:_N, 0:1] + 1.0
    dis = lax.rsqrt(deg)
    h = jnp.dot(x_ref[...], w_ref[...], preferred_element_type=jnp.float32)
    g_ref[...] = h * dis
    dis_ref[...] = dis


def _tc_prep(x, W, deg128):
    return pl.pallas_call(
        _prep_body,
        out_shape=[
            jax.ShapeDtypeStruct((_N, _D), jnp.float32),
            jax.ShapeDtypeStruct((_N, 1), jnp.float32),
        ],
    )(x, W, deg128)


def _final_body(agg_ref, g_ref, dis_ref, b_ref, gam_ref, bet_ref, o_ref):
    pre = (agg_ref[0, 0:_N, :] + agg_ref[1, 0:_N, :] + g_ref[...]) * dis_ref[...] + b_ref[...]
    mean = jnp.mean(pre, axis=0, keepdims=True)
    cen = pre - mean
    var = jnp.mean(cen * cen, axis=0, keepdims=True)
    o_ref[...] = jnp.maximum(
        cen * lax.rsqrt(var + 1e-5) * gam_ref[...] + bet_ref[...], 0.0)


def _tc_final(agg, g, dis, b, gamma, beta):
    return pl.pallas_call(
        _final_body,
        out_shape=jax.ShapeDtypeStruct((_N, _D), jnp.float32),
    )(agg, g, dis, b, gamma, beta)


def kernel(x, edge_index, W, b, gamma, beta):
    pad = _EP - _E
    src_pad = jnp.tile(jnp.arange(240, dtype=jnp.int32), pad // 240)
    dst_pad = _N + src_pad
    src = jnp.concatenate([edge_index[0], src_pad])
    dst = jnp.concatenate([edge_index[1], dst_pad])
    dst_r = dst.reshape(_NW, _NB, _B)
    pidx = (src | (dst << 16)).reshape(_NW, _NB, _B)
    zrows = jnp.zeros((_B, _D), jnp.float32)
    ones = jnp.ones((_B, _D), jnp.float32)
    deg128 = _sc_deg(dst_r, zrows, ones)
    g, dis = _tc_prep(x, W, deg128)
    agg = _sc_agg(g, pidx, zrows)
    out = _tc_final(agg, g, dis, b.reshape(1, _D), gamma.reshape(1, _D),
                    beta.reshape(1, _D))
    return out


# speedup vs baseline: 33.7431x; 1.1960x over previous
import functools

import jax
import jax.numpy as jnp
from jax import lax
from jax.experimental import pallas as pl
from jax.experimental.pallas import tpu as pltpu
from jax.experimental.pallas import tpu_sc as plsc

_N = 10000
_E = 320000
_D = 128
_NC = 2
_NS = 16
_NW = _NC * _NS
_B = 128
_NB = 80
_EP = _NW * _NB * _B
_NP = 10240
_RPW = _NP // _NS
_L = 16

_mesh = plsc.VectorSubcoreMesh(core_axis_name="c", subcore_axis_name="s")


_DW = 32


@functools.partial(
    pl.kernel,
    out_type=jax.ShapeDtypeStruct((_NC, _NP, _D), jnp.float32),
    mesh=_mesh,
    compiler_params=pltpu.CompilerParams(use_tc_tiling_on_sc=False),
    scratch_types=[
        pltpu.VMEM((_NB, _B), jnp.int32),
        pltpu.VMEM((_B, _DW), jnp.float32),
        pltpu.VMEM((_B, _DW), jnp.float32),
        pltpu.VMEM((_RPW, _DW), jnp.float32),
        pltpu.VMEM((_B, _D), jnp.float32),
        pltpu.VMEM_SHARED((_NP, _DW), jnp.float32),
        pltpu.SemaphoreType.DMA,
    ],
)
def _sc_deg(dst_hbm, deg_hbm, idx_v, ones_v, zero_v, nar_v, stg_v, deg_s, sem):
    cid = lax.axis_index("c")
    sid = lax.axis_index("s")
    wid = cid * _NS + sid
    pltpu.sync_copy(dst_hbm.at[wid], idx_v)
    one16 = jnp.ones((_L,), jnp.float32)
    zero16 = jnp.zeros((_L,), jnp.float32)

    def fill(r, carry):
        for k in range(_DW // _L):
            ones_v[r, pl.ds(k * _L, _L)] = one16
            zero_v[r, pl.ds(k * _L, _L)] = zero16
        return carry

    lax.fori_loop(0, _B, fill, 0)
    for k in range(_RPW // _B):
        pltpu.sync_copy(zero_v, deg_s.at[pl.ds(sid * _RPW + k * _B, _B)])
    plsc.subcore_barrier()

    def fire(j, carry):
        pltpu.async_copy(ones_v, deg_s.at[idx_v.at[j]], sem, add=True)
        return carry

    lax.fori_loop(0, _NB, fire, 0)

    def drain(j, carry):
        pltpu.make_async_copy(ones_v, deg_s.at[idx_v.at[j]], sem).wait()
        return carry

    lax.fori_loop(0, _NB, drain, 0)
    plsc.subcore_barrier()
    pltpu.sync_copy(deg_s.at[pl.ds(sid * _RPW, _RPW)], nar_v)
    for c in range(_RPW // _B):
        def rep(r, carry):
            v = nar_v[c * _B + r, pl.ds(0, _L)]
            for k in range(_D // _L):
                stg_v[r, pl.ds(k * _L, _L)] = v
            return carry

        lax.fori_loop(0, _B, rep, 0)
        pltpu.sync_copy(stg_v, deg_hbm.at[cid, pl.ds(sid * _RPW + c * _B, _B)])


@functools.partial(
    pl.kernel,
    out_type=jax.ShapeDtypeStruct((_NC, _NP, _D), jnp.float32),
    mesh=_mesh,
    scratch_types=[
        pltpu.VMEM((_NB, _B), jnp.int32),
        pltpu.VMEM((_B,), jnp.int32),
        pltpu.VMEM((_B,), jnp.int32),
        pltpu.VMEM((_B,), jnp.int32),
        pltpu.VMEM((_B,), jnp.int32),
        pltpu.VMEM((_B, _D), jnp.float32),
        pltpu.VMEM((_B, _D), jnp.float32),
        pltpu.VMEM_SHARED((_NP, _D), jnp.float32),
        pltpu.SemaphoreType.DMA,
        pltpu.SemaphoreType.DMA,
        pltpu.SemaphoreType.DMA,
        pltpu.SemaphoreType.DMA,
    ],
)
def _sc_agg(g_hbm, pidx_hbm, zrows_hbm, agg_hbm,
            pidx_v, si0, si1, di0, di1, rows0, rows1, agg_s,
            gs0, gs1, ss0, ss1):
    cid = lax.axis_index("c")
    sid = lax.axis_index("s")
    wid = cid * _NS + sid
    pltpu.sync_copy(pidx_hbm.at[wid], pidx_v)
    pltpu.sync_copy(zrows_hbm, rows0)
    for k in range(_RPW // _B):
        pltpu.sync_copy(rows0, agg_s.at[pl.ds(sid * _RPW + k * _B, _B)])
    plsc.subcore_barrier()

    def unpack(j, si, di):
        for k in range(_B // _L):
            p = pidx_v[j, pl.ds(k * _L, _L)]
            si[pl.ds(k * _L, _L)] = lax.bitwise_and(p, 0xFFFF)
            di[pl.ds(k * _L, _L)] = lax.shift_right_logical(p, 16)

    unpack(0, si0, di0)
    pltpu.async_copy(g_hbm.at[si0], rows0, gs0)
    unpack(1, si1, di1)
    pltpu.async_copy(g_hbm.at[si1], rows1, gs1)

    def step(i, carry):
        j = 2 * i
        pltpu.make_async_copy(g_hbm.at[si0], rows0, gs0).wait()
        sc0 = pltpu.async_copy(rows0, agg_s.at[di0], ss0, add=True)
        pltpu.make_async_copy(g_hbm.at[si1], rows1, gs1).wait()
        sc1 = pltpu.async_copy(rows1, agg_s.at[di1], ss1, add=True)
        sc0.wait()

        @pl.when(j + 2 < _NB)
        def _():
            unpack(j + 2, si0, di0)
            pltpu.async_copy(g_hbm.at[si0], rows0, gs0)

        sc1.wait()

        @pl.when(j + 3 < _NB)
        def _():
            unpack(j + 3, si1, di1)
            pltpu.async_copy(g_hbm.at[si1], rows1, gs1)

        return carry

    lax.fori_loop(0, _NB // 2, step, 0)
    plsc.subcore_barrier()
    for k in range(_RPW // _B):
        pltpu.sync_copy(agg_s.at[pl.ds(sid * _RPW + k * _B, _B)], rows0)
        pltpu.sync_copy(rows0, agg_hbm.at[cid, pl.ds(sid * _RPW + k * _B, _B)])


def _prep_body(x_ref, w_ref, deg_ref, g_ref, dis_ref):
    deg = deg_ref[0, 0:_N, 0:1] + deg_ref[1, 0:_N, 0:1] + 1.0
    dis = lax.rsqrt(deg)
    h = jnp.dot(x_ref[...], w_ref[...], preferred_element_type=jnp.float32)
    g_ref[...] = h * dis
    dis_ref[...] = dis


def _tc_prep(x, W, deg128):
    return pl.pallas_call(
        _prep_body,
        out_shape=[
            jax.ShapeDtypeStruct((_N, _D), jnp.float32),
            jax.ShapeDtypeStruct((_N, 1), jnp.float32),
        ],
    )(x, W, deg128)


def _final_body(agg_ref, g_ref, dis_ref, b_ref, gam_ref, bet_ref, o_ref):
    pre = (agg_ref[0, 0:_N, :] + agg_ref[1, 0:_N, :] + g_ref[...]) * dis_ref[...] + b_ref[...]
    mean = jnp.mean(pre, axis=0, keepdims=True)
    cen = pre - mean
    var = jnp.mean(cen * cen, axis=0, keepdims=True)
    o_ref[...] = jnp.maximum(
        cen * lax.rsqrt(var + 1e-5) * gam_ref[...] + bet_ref[...], 0.0)


def _tc_final(agg, g, dis, b, gamma, beta):
    return pl.pallas_call(
        _final_body,
        out_shape=jax.ShapeDtypeStruct((_N, _D), jnp.float32),
    )(agg, g, dis, b, gamma, beta)


def kernel(x, edge_index, W, b, gamma, beta):
    pad = _EP - _E
    src_pad = jnp.tile(jnp.arange(240, dtype=jnp.int32), pad // 240)
    dst_pad = _N + src_pad
    src = jnp.concatenate([edge_index[0], src_pad])
    dst = jnp.concatenate([edge_index[1], dst_pad])
    dst_r = dst.reshape(_NW, _NB, _B)
    pidx = (src | (dst << 16)).reshape(_NW, _NB, _B)
    zrows = jnp.zeros((_B, _D), jnp.float32)
    deg128 = _sc_deg(dst_r)
    g, dis = _tc_prep(x, W, deg128)
    agg = _sc_agg(g, pidx, zrows)
    out = _tc_final(agg, g, dis, b.reshape(1, _D), gamma.reshape(1, _D),
                    beta.reshape(1, _D))
    return out


# speedup vs baseline: 34.8610x; 1.0331x over previous
import functools

import jax
import jax.numpy as jnp
from jax import lax
from jax.experimental import pallas as pl
from jax.experimental.pallas import tpu as pltpu
from jax.experimental.pallas import tpu_sc as plsc

_N = 10000
_E = 320000
_D = 128
_NC = 2
_NS = 16
_NW = _NC * _NS
_B = 128
_NB = 80
_EP = _NW * _NB * _B
_NP = 10240
_RPW = _NP // _NS
_L = 16

_mesh = plsc.VectorSubcoreMesh(core_axis_name="c", subcore_axis_name="s")


_DW = 16


@functools.partial(
    pl.kernel,
    out_type=jax.ShapeDtypeStruct((_NC, _NP, _D), jnp.float32),
    mesh=_mesh,
    compiler_params=pltpu.CompilerParams(use_tc_tiling_on_sc=False),
    scratch_types=[
        pltpu.VMEM((_NB, _B), jnp.int32),
        pltpu.VMEM((_B, _DW), jnp.float32),
        pltpu.VMEM((_B, _DW), jnp.float32),
        pltpu.VMEM((_RPW, _DW), jnp.float32),
        pltpu.VMEM((_B, _D), jnp.float32),
        pltpu.VMEM_SHARED((_NP, _DW), jnp.float32),
        pltpu.SemaphoreType.DMA,
    ],
)
def _sc_deg(dst_hbm, deg_hbm, idx_v, ones_v, zero_v, nar_v, stg_v, deg_s, sem):
    cid = lax.axis_index("c")
    sid = lax.axis_index("s")
    wid = cid * _NS + sid
    pltpu.sync_copy(dst_hbm.at[wid], idx_v)
    one16 = jnp.ones((_L,), jnp.float32)
    zero16 = jnp.zeros((_L,), jnp.float32)

    def fill(r, carry):
        for k in range(_DW // _L):
            ones_v[r, pl.ds(k * _L, _L)] = one16
            zero_v[r, pl.ds(k * _L, _L)] = zero16
        return carry

    lax.fori_loop(0, _B, fill, 0)
    for k in range(_RPW // _B):
        pltpu.sync_copy(zero_v, deg_s.at[pl.ds(sid * _RPW + k * _B, _B)])
    plsc.subcore_barrier()

    def fire(j, carry):
        pltpu.async_copy(ones_v, deg_s.at[idx_v.at[j]], sem, add=True)
        return carry

    lax.fori_loop(0, _NB, fire, 0)

    def drain(j, carry):
        pltpu.make_async_copy(ones_v, deg_s.at[idx_v.at[j]], sem).wait()
        return carry

    lax.fori_loop(0, _NB, drain, 0)
    plsc.subcore_barrier()
    pltpu.sync_copy(deg_s.at[pl.ds(sid * _RPW, _RPW)], nar_v)
    for c in range(_RPW // _B):
        def rep(r, carry):
            v = nar_v[c * _B + r, pl.ds(0, _L)]
            for k in range(_D // _L):
                stg_v[r, pl.ds(k * _L, _L)] = v
            return carry

        lax.fori_loop(0, _B, rep, 0)
        pltpu.sync_copy(stg_v, deg_hbm.at[cid, pl.ds(sid * _RPW + c * _B, _B)])


@functools.partial(
    pl.kernel,
    out_type=jax.ShapeDtypeStruct((_NC, _NP, _D), jnp.float32),
    mesh=_mesh,
    scratch_types=[
        pltpu.VMEM((_NB, _B), jnp.int32),
        pltpu.VMEM((_B,), jnp.int32),
        pltpu.VMEM((_B,), jnp.int32),
        pltpu.VMEM((_B,), jnp.int32),
        pltpu.VMEM((_B,), jnp.int32),
        pltpu.VMEM((_B, _D), jnp.float32),
        pltpu.VMEM((_B, _D), jnp.float32),
        pltpu.VMEM_SHARED((_NP, _D), jnp.float32),
        pltpu.SemaphoreType.DMA,
        pltpu.SemaphoreType.DMA,
        pltpu.SemaphoreType.DMA,
        pltpu.SemaphoreType.DMA,
    ],
)
def _sc_agg(g_hbm, pidx_hbm, zrows_hbm, agg_hbm,
            pidx_v, si0, si1, di0, di1, rows0, rows1, agg_s,
            gs0, gs1, ss0, ss1):
    cid = lax.axis_index("c")
    sid = lax.axis_index("s")
    wid = cid * _NS + sid
    pltpu.sync_copy(pidx_hbm.at[wid], pidx_v)
    pltpu.sync_copy(zrows_hbm, rows0)
    for k in range(_RPW // _B):
        pltpu.sync_copy(rows0, agg_s.at[pl.ds(sid * _RPW + k * _B, _B)])
    plsc.subcore_barrier()

    def unpack(j, si, di):
        for k in range(_B // _L):
            p = pidx_v[j, pl.ds(k * _L, _L)]
            si[pl.ds(k * _L, _L)] = lax.bitwise_and(p, 0xFFFF)
            di[pl.ds(k * _L, _L)] = lax.shift_right_logical(p, 16)

    unpack(0, si0, di0)
    pltpu.async_copy(g_hbm.at[si0], rows0, gs0)
    unpack(1, si1, di1)
    pltpu.async_copy(g_hbm.at[si1], rows1, gs1)

    def step(i, carry):
        j = 2 * i
        pltpu.make_async_copy(g_hbm.at[si0], rows0, gs0).wait()
        sc0 = pltpu.async_copy(rows0, agg_s.at[di0], ss0, add=True)
        pltpu.make_async_copy(g_hbm.at[si1], rows1, gs1).wait()
        sc1 = pltpu.async_copy(rows1, agg_s.at[di1], ss1, add=True)
        sc0.wait()

        @pl.when(j + 2 < _NB)
        def _():
            unpack(j + 2, si0, di0)
            pltpu.async_copy(g_hbm.at[si0], rows0, gs0)

        sc1.wait()

        @pl.when(j + 3 < _NB)
        def _():
            unpack(j + 3, si1, di1)
            pltpu.async_copy(g_hbm.at[si1], rows1, gs1)

        return carry

    lax.fori_loop(0, _NB // 2, step, 0)
    plsc.subcore_barrier()
    for k in range(_RPW // _B):
        pltpu.sync_copy(agg_s.at[pl.ds(sid * _RPW + k * _B, _B)], rows0)
        pltpu.sync_copy(rows0, agg_hbm.at[cid, pl.ds(sid * _RPW + k * _B, _B)])


def _prep_body(x_ref, w_ref, deg_ref, g_ref, dis_ref):
    deg = deg_ref[0, 0:_N, 0:1] + deg_ref[1, 0:_N, 0:1] + 1.0
    dis = lax.rsqrt(deg)
    h = jnp.dot(x_ref[...], w_ref[...], preferred_element_type=jnp.float32)
    g_ref[...] = h * dis
    dis_ref[...] = dis


def _tc_prep(x, W, deg128):
    return pl.pallas_call(
        _prep_body,
        out_shape=[
            jax.ShapeDtypeStruct((_N, _D), jnp.float32),
            jax.ShapeDtypeStruct((_N, 1), jnp.float32),
        ],
    )(x, W, deg128)


def _final_body(agg_ref, g_ref, dis_ref, b_ref, gam_ref, bet_ref, o_ref):
    pre = (agg_ref[0, 0:_N, :] + agg_ref[1, 0:_N, :] + g_ref[...]) * dis_ref[...] + b_ref[...]
    mean = jnp.mean(pre, axis=0, keepdims=True)
    cen = pre - mean
    var = jnp.mean(cen * cen, axis=0, keepdims=True)
    o_ref[...] = jnp.maximum(
        cen * lax.rsqrt(var + 1e-5) * gam_ref[...] + bet_ref[...], 0.0)


def _tc_final(agg, g, dis, b, gamma, beta):
    return pl.pallas_call(
        _final_body,
        out_shape=jax.ShapeDtypeStruct((_N, _D), jnp.float32),
    )(agg, g, dis, b, gamma, beta)


def kernel(x, edge_index, W, b, gamma, beta):
    pad = _EP - _E
    src_pad = jnp.tile(jnp.arange(240, dtype=jnp.int32), pad // 240)
    dst_pad = _N + src_pad
    src = jnp.concatenate([edge_index[0], src_pad])
    dst = jnp.concatenate([edge_index[1], dst_pad])
    dst_r = dst.reshape(_NW, _NB, _B)
    pidx = (src | (dst << 16)).reshape(_NW, _NB, _B)
    zrows = jnp.zeros((_B, _D), jnp.float32)
    deg128 = _sc_deg(dst_r)
    g, dis = _tc_prep(x, W, deg128)
    agg = _sc_agg(g, pidx, zrows)
    out = _tc_final(agg, g, dis, b.reshape(1, _D), gamma.reshape(1, _D),
                    beta.reshape(1, _D))
    return out


# speedup vs baseline: 34.9042x; 1.0012x over previous
import functools

import jax
import jax.numpy as jnp
from jax import lax
from jax.experimental import pallas as pl
from jax.experimental.pallas import tpu as pltpu
from jax.experimental.pallas import tpu_sc as plsc

_N = 10000
_E = 320000
_D = 128
_NC = 2
_NS = 16
_NW = _NC * _NS
_B = 128
_NB = 80
_EP = _NW * _NB * _B
_NP = 10240
_RPW = _NP // _NS
_L = 16

_mesh = plsc.VectorSubcoreMesh(core_axis_name="c", subcore_axis_name="s")


_DW = 16


@functools.partial(
    pl.kernel,
    out_type=jax.ShapeDtypeStruct((_NC, _NP, _D), jnp.float32),
    mesh=_mesh,
    compiler_params=pltpu.CompilerParams(use_tc_tiling_on_sc=False),
    scratch_types=[
        pltpu.VMEM((_NB, _B), jnp.int32),
        pltpu.VMEM((_B, _DW), jnp.float32),
        pltpu.VMEM((_B, _DW), jnp.float32),
        pltpu.VMEM((_RPW, _DW), jnp.float32),
        pltpu.VMEM((_B, _D), jnp.float32),
        pltpu.VMEM_SHARED((_NP, _DW), jnp.float32),
        pltpu.SemaphoreType.DMA,
    ],
)
def _sc_deg(dst_hbm, deg_hbm, idx_v, ones_v, zero_v, nar_v, stg_v, deg_s, sem):
    cid = lax.axis_index("c")
    sid = lax.axis_index("s")
    wid = cid * _NS + sid
    pltpu.sync_copy(dst_hbm.at[wid], idx_v)
    one16 = jnp.ones((_L,), jnp.float32)
    zero16 = jnp.zeros((_L,), jnp.float32)

    def fill(r, carry):
        for k in range(_DW // _L):
            ones_v[r, pl.ds(k * _L, _L)] = one16
            zero_v[r, pl.ds(k * _L, _L)] = zero16
        return carry

    lax.fori_loop(0, _B, fill, 0)
    for k in range(_RPW // _B):
        pltpu.sync_copy(zero_v, deg_s.at[pl.ds(sid * _RPW + k * _B, _B)])
    plsc.subcore_barrier()

    def fire(j, carry):
        pltpu.async_copy(ones_v, deg_s.at[idx_v.at[j]], sem, add=True)
        return carry

    lax.fori_loop(0, _NB, fire, 0)

    def drain(j, carry):
        pltpu.make_async_copy(ones_v, deg_s.at[idx_v.at[j]], sem).wait()
        return carry

    lax.fori_loop(0, _NB, drain, 0)
    plsc.subcore_barrier()
    pltpu.sync_copy(deg_s.at[pl.ds(sid * _RPW, _RPW)], nar_v)
    for c in range(_RPW // _B):
        def rep(r, carry):
            v = nar_v[c * _B + r, pl.ds(0, _L)]
            for k in range(_D // _L):
                stg_v[r, pl.ds(k * _L, _L)] = v
            return carry

        lax.fori_loop(0, _B, rep, 0)
        pltpu.sync_copy(stg_v, deg_hbm.at[cid, pl.ds(sid * _RPW + c * _B, _B)])


@functools.partial(
    pl.kernel,
    out_type=jax.ShapeDtypeStruct((_NC, _NP, _D), jnp.float32),
    mesh=_mesh,
    compiler_params=pltpu.CompilerParams(use_tc_tiling_on_sc=False),
    scratch_types=[
        pltpu.VMEM((_NB, _B), jnp.int32),
        pltpu.VMEM((_B,), jnp.int32),
        pltpu.VMEM((_B,), jnp.int32),
        pltpu.VMEM((_B,), jnp.int32),
        pltpu.VMEM((_B,), jnp.int32),
        pltpu.VMEM((_B, _D), jnp.float32),
        pltpu.VMEM((_B, _D), jnp.float32),
        pltpu.VMEM_SHARED((_NP, _D), jnp.float32),
        pltpu.SemaphoreType.DMA,
        pltpu.SemaphoreType.DMA,
        pltpu.SemaphoreType.DMA,
        pltpu.SemaphoreType.DMA,
    ],
)
def _sc_agg(g_hbm, pidx_hbm, zrows_hbm, agg_hbm,
            pidx_v, si0, si1, di0, di1, rows0, rows1, agg_s,
            gs0, gs1, ss0, ss1):
    cid = lax.axis_index("c")
    sid = lax.axis_index("s")
    wid = cid * _NS + sid
    pltpu.sync_copy(pidx_hbm.at[wid], pidx_v)
    pltpu.sync_copy(zrows_hbm, rows0)
    for k in range(_RPW // _B):
        pltpu.sync_copy(rows0, agg_s.at[pl.ds(sid * _RPW + k * _B, _B)])
    plsc.subcore_barrier()

    def unpack(j, si, di):
        for k in range(_B // _L):
            p = pidx_v[j, pl.ds(k * _L, _L)]
            si[pl.ds(k * _L, _L)] = lax.bitwise_and(p, 0xFFFF)
            di[pl.ds(k * _L, _L)] = lax.shift_right_logical(p, 16)

    unpack(0, si0, di0)
    pltpu.async_copy(g_hbm.at[si0], rows0, gs0)
    unpack(1, si1, di1)
    pltpu.async_copy(g_hbm.at[si1], rows1, gs1)

    def step(i, carry):
        j = 2 * i
        pltpu.make_async_copy(g_hbm.at[si0], rows0, gs0).wait()
        sc0 = pltpu.async_copy(rows0, agg_s.at[di0], ss0, add=True)
        pltpu.make_async_copy(g_hbm.at[si1], rows1, gs1).wait()
        sc1 = pltpu.async_copy(rows1, agg_s.at[di1], ss1, add=True)
        sc0.wait()

        @pl.when(j + 2 < _NB)
        def _():
            unpack(j + 2, si0, di0)
            pltpu.async_copy(g_hbm.at[si0], rows0, gs0)

        sc1.wait()

        @pl.when(j + 3 < _NB)
        def _():
            unpack(j + 3, si1, di1)
            pltpu.async_copy(g_hbm.at[si1], rows1, gs1)

        return carry

    lax.fori_loop(0, _NB // 2, step, 0)
    plsc.subcore_barrier()
    for k in range(_RPW // _B):
        pltpu.sync_copy(agg_s.at[pl.ds(sid * _RPW + k * _B, _B)], rows0)
        pltpu.sync_copy(rows0, agg_hbm.at[cid, pl.ds(sid * _RPW + k * _B, _B)])


def _prep_body(x_ref, w_ref, deg_ref, g_ref, dis_ref):
    deg = deg_ref[0, 0:_N, 0:1] + deg_ref[1, 0:_N, 0:1] + 1.0
    dis = lax.rsqrt(deg)
    h = jnp.dot(x_ref[...], w_ref[...], preferred_element_type=jnp.float32)
    g_ref[...] = h * dis
    dis_ref[...] = dis


def _tc_prep(x, W, deg128):
    return pl.pallas_call(
        _prep_body,
        out_shape=[
            jax.ShapeDtypeStruct((_N, _D), jnp.float32),
            jax.ShapeDtypeStruct((_N, 1), jnp.float32),
        ],
    )(x, W, deg128)


def _final_body(agg_ref, g_ref, dis_ref, b_ref, gam_ref, bet_ref, o_ref):
    pre = (agg_ref[0, 0:_N, :] + agg_ref[1, 0:_N, :] + g_ref[...]) * dis_ref[...] + b_ref[...]
    mean = jnp.mean(pre, axis=0, keepdims=True)
    cen = pre - mean
    var = jnp.mean(cen * cen, axis=0, keepdims=True)
    o_ref[...] = jnp.maximum(
        cen * lax.rsqrt(var + 1e-5) * gam_ref[...] + bet_ref[...], 0.0)


def _tc_final(agg, g, dis, b, gamma, beta):
    return pl.pallas_call(
        _final_body,
        out_shape=jax.ShapeDtypeStruct((_N, _D), jnp.float32),
    )(agg, g, dis, b, gamma, beta)


def kernel(x, edge_index, W, b, gamma, beta):
    pad = _EP - _E
    src_pad = jnp.tile(jnp.arange(240, dtype=jnp.int32), pad // 240)
    dst_pad = _N + src_pad
    src = jnp.concatenate([edge_index[0], src_pad])
    dst = jnp.concatenate([edge_index[1], dst_pad])
    dst_r = dst.reshape(_NW, _NB, _B)
    pidx = (src | (dst << 16)).reshape(_NW, _NB, _B)
    zrows = jnp.zeros((_B, _D), jnp.float32)
    deg128 = _sc_deg(dst_r)
    g, dis = _tc_prep(x, W, deg128)
    agg = _sc_agg(g, pidx, zrows)
    out = _tc_final(agg, g, dis, b.reshape(1, _D), gamma.reshape(1, _D),
                    beta.reshape(1, _D))
    return out
